# baseline (speedup 1.0000x reference)
import functools

import jax
import jax.numpy as jnp
from jax.experimental import pallas as pl
from jax.experimental.pallas import tpu as pltpu

_CHUNK = 56


def _clamped_chunk(l, c, ch):
    cmax = jnp.maximum((l + ch - 1) // ch - 1, 0)
    return jnp.minimum(c, cmax)


def _pool_body(seg_ref, x_ref, out_ref, *, ch):
    b = pl.program_id(0)
    s = pl.program_id(1)
    c = pl.program_id(2)
    l = seg_ref[b, s]
    x = x_ref[0, 0]
    rows = jax.lax.broadcasted_iota(jnp.int32, (ch, 1), 0) + c * ch
    xm = jnp.where(rows < l, x, 0.0)
    part = jnp.sum(xm, axis=0, keepdims=True)[None]

    @pl.when(jnp.logical_and(s == 0, c == 0))
    def _init():
        out_ref[...] = part

    @pl.when(jnp.logical_or(s != 0, c != 0))
    def _acc():
        out_ref[...] += part


def _mlp_body(segf_ref, pooled_ref, c1_ref, c2_ref, Wenc_ref, benc_ref,
              Wt1_ref, bt1_ref, Wt2_ref, bt2_ref, temb_ref, Wc_ref, bc_ref,
              q_ref, tp_ref):
    TE = temb_ref.shape[1]
    count = jnp.sum(segf_ref[...], axis=1, keepdims=True)
    pooled = pooled_ref[...] / count
    vf = jax.nn.relu(
        jnp.dot(pooled, Wenc_ref[...], preferred_element_type=jnp.float32)
        + benc_ref[...])
    cat = jnp.concatenate([c1_ref[...], c2_ref[...], vf], axis=1)
    h = jax.nn.relu(
        jnp.dot(cat, Wt1_ref[...], preferred_element_type=jnp.float32)
        + bt1_ref[...])
    logits = jnp.dot(h, Wt2_ref[...], preferred_element_type=jnp.float32) + bt2_ref[...]
    m = jnp.max(logits, axis=1, keepdims=True)
    e = jnp.exp(logits - m)
    tp_ref[...] = (e / jnp.sum(e, axis=1, keepdims=True))[:, None, :]
    E_T = jax.lax.dot_general(Wc_ref[0:TE, :], temb_ref[...],
                              dimension_numbers=(((0,), (1,)), ((), ())),
                              preferred_element_type=jnp.float32)
    catv = jnp.concatenate([vf, c1_ref[...], c2_ref[...]], axis=1)
    V = jnp.dot(catv, Wc_ref[TE:, :], preferred_element_type=jnp.float32) + bc_ref[...]
    q_ref[...] = jax.nn.relu(V[:, :, None] + E_T[None, :, :])


def _score_body(seg_ref, x_ref, q_ref, tp_ref, out_ref, *, tn, ch):
    b = pl.program_id(0)
    s = pl.program_id(1)
    c = pl.program_id(2)
    l = seg_ref[b, s]
    x = x_ref[0, 0]
    q = q_ref[0]
    L = jnp.dot(x, q, preferred_element_type=jnp.float32)
    sc = jax.nn.sigmoid(L) * tp_ref[0]
    sc = jax.nn.relu(sc - 0.01)
    tot = jnp.sum(sc, axis=1, keepdims=True) * (1.0 / tn)
    rows = jax.lax.broadcasted_iota(jnp.int32, tot.shape, 0) + c * ch
    tot = jnp.where(rows < l, tot, 0.0)
    out_ref[...] = tot[None, None]


def kernel(batch, seg_len, concept1, concept2, W_enc, b_enc, W_t1, b_t1,
           W_t2, b_t2, topic_emb, W_c, b_c):
    B, S, F, D = batch.shape
    TN, TE = topic_emb.shape
    SH = W_enc.shape[1]
    CH = _CHUNK
    C = -(-F // CH)

    seg_len = seg_len.astype(jnp.int32)

    sums = pl.pallas_call(
        functools.partial(_pool_body, ch=CH),
        grid_spec=pltpu.PrefetchScalarGridSpec(
            num_scalar_prefetch=1,
            grid=(B, S, C),
            in_specs=[pl.BlockSpec(
                (1, 1, CH, D),
                lambda b, s, c, seg: (b, s, _clamped_chunk(seg[b, s], c, CH), 0))],
            out_specs=pl.BlockSpec((1, 1, D), lambda b, s, c, seg: (b, 0, 0)),
        ),
        out_shape=jax.ShapeDtypeStruct((B, 1, D), jnp.float32),
        compiler_params=pltpu.CompilerParams(
            dimension_semantics=("parallel", "arbitrary", "arbitrary")),
    )(seg_len, batch)

    q, tp = pl.pallas_call(
        _mlp_body,
        out_shape=(jax.ShapeDtypeStruct((B, D, TN), jnp.float32),
                   jax.ShapeDtypeStruct((B, 1, TN), jnp.float32)),
    )(seg_len.astype(jnp.float32), sums.reshape(B, D), concept1, concept2,
      W_enc, b_enc.reshape(1, SH), W_t1, b_t1.reshape(1, -1),
      W_t2, b_t2.reshape(1, TN), topic_emb, W_c, b_c.reshape(1, D))

    overall = pl.pallas_call(
        functools.partial(_score_body, tn=float(TN), ch=CH),
        grid_spec=pltpu.PrefetchScalarGridSpec(
            num_scalar_prefetch=1,
            grid=(B, S, C),
            in_specs=[
                pl.BlockSpec(
                    (1, 1, CH, D),
                    lambda b, s, c, seg: (b, s, _clamped_chunk(seg[b, s], c, CH), 0)),
                pl.BlockSpec((1, D, TN), lambda b, s, c, seg: (b, 0, 0)),
                pl.BlockSpec((1, 1, TN), lambda b, s, c, seg: (b, 0, 0)),
            ],
            out_specs=pl.BlockSpec((1, 1, CH, 1), lambda b, s, c, seg: (b, s, c, 0)),
        ),
        out_shape=jax.ShapeDtypeStruct((B, S, F, 1), jnp.float32),
        compiler_params=pltpu.CompilerParams(
            dimension_semantics=("parallel", "arbitrary", "arbitrary")),
    )(seg_len, batch, q, tp)

    overall = overall.reshape(B, S, F)
    return (overall, overall)


# speedup vs baseline: 6.1950x; 6.1950x over previous
import functools

import jax
import jax.numpy as jnp
from jax.experimental import pallas as pl
from jax.experimental.pallas import tpu as pltpu


def _seg_block(n_seg):
    for sb in (10, 5, 4, 2):
        if n_seg % sb == 0:
            return sb
    return 1


def _pool_body(seg_ref, x_ref, out_ref, *, sb, f):
    b = pl.program_id(0)
    g = pl.program_id(1)
    x = x_ref[0].reshape(sb * f, x_ref.shape[-1])
    f_lane = jax.lax.rem(jax.lax.broadcasted_iota(jnp.int32, (1, sb * f), 1), f)
    l_lane = jnp.concatenate(
        [jnp.full((1, f), seg_ref[b, g * sb + j], jnp.int32) for j in range(sb)],
        axis=1)
    mask = (f_lane < l_lane).astype(jnp.float32)
    part = jnp.dot(mask, x, preferred_element_type=jnp.float32)[None]

    @pl.when(g == 0)
    def _init():
        out_ref[...] = part

    @pl.when(g != 0)
    def _acc():
        out_ref[...] += part


def _mlp_body(segf_ref, pooled_ref, c1_ref, c2_ref, Wenc_ref, benc_ref,
              Wt1_ref, bt1_ref, Wt2_ref, bt2_ref, temb_ref, Wc_ref, bc_ref,
              q_ref, tp_ref):
    TE = temb_ref.shape[1]
    count = jnp.sum(segf_ref[...], axis=1, keepdims=True)
    pooled = pooled_ref[...] / count
    vf = jax.nn.relu(
        jnp.dot(pooled, Wenc_ref[...], preferred_element_type=jnp.float32)
        + benc_ref[...])
    cat = jnp.concatenate([c1_ref[...], c2_ref[...], vf], axis=1)
    h = jax.nn.relu(
        jnp.dot(cat, Wt1_ref[...], preferred_element_type=jnp.float32)
        + bt1_ref[...])
    logits = jnp.dot(h, Wt2_ref[...], preferred_element_type=jnp.float32) + bt2_ref[...]
    m = jnp.max(logits, axis=1, keepdims=True)
    e = jnp.exp(logits - m)
    tp_ref[...] = (e / jnp.sum(e, axis=1, keepdims=True))[:, None, :]
    E_T = jax.lax.dot_general(Wc_ref[0:TE, :], temb_ref[...],
                              dimension_numbers=(((0,), (1,)), ((), ())),
                              preferred_element_type=jnp.float32)
    catv = jnp.concatenate([vf, c1_ref[...], c2_ref[...]], axis=1)
    V = jnp.dot(catv, Wc_ref[TE:, :], preferred_element_type=jnp.float32) + bc_ref[...]
    q_ref[...] = jax.nn.relu(V[:, :, None] + E_T[None, :, :])


def _score_body(seg_ref, x_ref, q_ref, tp_ref, out_ref, *, tn, sb, f):
    b = pl.program_id(0)
    g = pl.program_id(1)
    x = x_ref[0].reshape(sb * f, x_ref.shape[-1])
    q = q_ref[0]
    L = jnp.dot(x, q, preferred_element_type=jnp.float32)
    sc = jax.nn.sigmoid(L) * tp_ref[0]
    sc = jax.nn.relu(sc - 0.01)
    tot = jnp.sum(sc, axis=1, keepdims=True) * (1.0 / tn)
    f_sub = jax.lax.rem(jax.lax.broadcasted_iota(jnp.int32, (sb * f, 1), 0), f)
    l_sub = jnp.concatenate(
        [jnp.full((f, 1), seg_ref[b, g * sb + j], jnp.int32) for j in range(sb)],
        axis=0)
    tot = jnp.where(f_sub < l_sub, tot, 0.0)
    out_ref[...] = tot.reshape(1, sb, f, 1)


def kernel(batch, seg_len, concept1, concept2, W_enc, b_enc, W_t1, b_t1,
           W_t2, b_t2, topic_emb, W_c, b_c):
    B, S, F, D = batch.shape
    TN, TE = topic_emb.shape
    SH = W_enc.shape[1]
    SB = _seg_block(S)
    G = S // SB

    seg_len = seg_len.astype(jnp.int32)

    sums = pl.pallas_call(
        functools.partial(_pool_body, sb=SB, f=F),
        grid_spec=pltpu.PrefetchScalarGridSpec(
            num_scalar_prefetch=1,
            grid=(B, G),
            in_specs=[pl.BlockSpec((1, SB, F, D), lambda b, g, seg: (b, g, 0, 0))],
            out_specs=pl.BlockSpec((1, 1, D), lambda b, g, seg: (b, 0, 0)),
        ),
        out_shape=jax.ShapeDtypeStruct((B, 1, D), jnp.float32),
        compiler_params=pltpu.CompilerParams(
            dimension_semantics=("parallel", "arbitrary")),
    )(seg_len, batch)

    q, tp = pl.pallas_call(
        _mlp_body,
        out_shape=(jax.ShapeDtypeStruct((B, D, TN), jnp.float32),
                   jax.ShapeDtypeStruct((B, 1, TN), jnp.float32)),
    )(seg_len.astype(jnp.float32), sums.reshape(B, D), concept1, concept2,
      W_enc, b_enc.reshape(1, SH), W_t1, b_t1.reshape(1, -1),
      W_t2, b_t2.reshape(1, TN), topic_emb, W_c, b_c.reshape(1, D))

    overall = pl.pallas_call(
        functools.partial(_score_body, tn=float(TN), sb=SB, f=F),
        grid_spec=pltpu.PrefetchScalarGridSpec(
            num_scalar_prefetch=1,
            grid=(B, G),
            in_specs=[
                pl.BlockSpec((1, SB, F, D), lambda b, g, seg: (b, g, 0, 0)),
                pl.BlockSpec((1, D, TN), lambda b, g, seg: (b, 0, 0)),
                pl.BlockSpec((1, 1, TN), lambda b, g, seg: (b, 0, 0)),
            ],
            out_specs=pl.BlockSpec((1, SB, F, 1), lambda b, g, seg: (b, g, 0, 0)),
        ),
        out_shape=jax.ShapeDtypeStruct((B, S, F, 1), jnp.float32),
        compiler_params=pltpu.CompilerParams(
            dimension_semantics=("parallel", "arbitrary")),
    )(seg_len, batch, q, tp)

    overall = overall.reshape(B, S, F)
    return (overall, overall)


# speedup vs baseline: 7.3295x; 1.1831x over previous
import functools

import jax
import jax.numpy as jnp
from jax.experimental import pallas as pl
from jax.experimental.pallas import tpu as pltpu


def _fused_body(seg_ref, x_ref, c1_ref, c2_ref, Wenc_ref, benc_ref,
                Wt1_ref, bt1_ref, Wt2_ref, bt2_ref, temb_ref, Wc_ref, bc_ref,
                out_ref, q_scr, tp_scr, *, s, f, tn):
    b = pl.program_id(0)
    p = pl.program_id(1)
    d = x_ref.shape[-1]

    @pl.when(p == 0)
    def _pool_and_mlp():
        TE = temb_ref.shape[1]
        x = x_ref[0].reshape(s * f, d)
        f_lane = jax.lax.rem(
            jax.lax.broadcasted_iota(jnp.int32, (1, s * f), 1), f)
        l_lane = jnp.concatenate(
            [jnp.full((1, f), seg_ref[b, j], jnp.int32) for j in range(s)],
            axis=1)
        mask = (f_lane < l_lane).astype(jnp.float32)
        sums = jnp.dot(mask, x, preferred_element_type=jnp.float32)
        count = jnp.sum(l_lane.astype(jnp.float32)) * (1.0 / f)
        pooled = sums / count
        vf = jax.nn.relu(
            jnp.dot(pooled, Wenc_ref[...], preferred_element_type=jnp.float32)
            + benc_ref[...])
        cat = jnp.concatenate([c1_ref[0], c2_ref[0], vf], axis=1)
        h = jax.nn.relu(
            jnp.dot(cat, Wt1_ref[...], preferred_element_type=jnp.float32)
            + bt1_ref[...])
        logits = (jnp.dot(h, Wt2_ref[...], preferred_element_type=jnp.float32)
                  + bt2_ref[...])
        m = jnp.max(logits, axis=1, keepdims=True)
        e = jnp.exp(logits - m)
        tp_scr[...] = e / jnp.sum(e, axis=1, keepdims=True)
        E_T = jax.lax.dot_general(Wc_ref[0:TE, :], temb_ref[...],
                                  dimension_numbers=(((0,), (1,)), ((), ())),
                                  preferred_element_type=jnp.float32)
        catv = jnp.concatenate([vf, c1_ref[0], c2_ref[0]], axis=1)
        V = jax.lax.dot_general(Wc_ref[TE:, :], catv,
                                dimension_numbers=(((0,), (1,)), ((), ())),
                                preferred_element_type=jnp.float32)
        q_scr[...] = jax.nn.relu(E_T + V + bc_ref[...])

    @pl.when(p == 1)
    def _score():
        x = x_ref[0].reshape(s * f, d)
        L = jnp.dot(x, q_scr[...], preferred_element_type=jnp.float32)
        sc = jax.nn.sigmoid(L) * tp_scr[...]
        sc = jax.nn.relu(sc - 0.01)
        tot = jnp.sum(sc, axis=1, keepdims=True) * (1.0 / tn)
        f_sub = jax.lax.rem(
            jax.lax.broadcasted_iota(jnp.int32, (s * f, 1), 0), f)
        l_sub = jnp.concatenate(
            [jnp.full((f, 1), seg_ref[b, j], jnp.int32) for j in range(s)],
            axis=0)
        tot = jnp.where(f_sub < l_sub, tot, 0.0)
        out_ref[...] = tot.reshape(1, s, f, 1)


def kernel(batch, seg_len, concept1, concept2, W_enc, b_enc, W_t1, b_t1,
           W_t2, b_t2, topic_emb, W_c, b_c):
    B, S, F, D = batch.shape
    TN, TE = topic_emb.shape
    SH = W_enc.shape[1]
    CD = concept1.shape[1]

    seg_len = seg_len.astype(jnp.int32)

    const = lambda *idx: (lambda b, p, seg: idx)

    overall = pl.pallas_call(
        functools.partial(_fused_body, s=S, f=F, tn=float(TN)),
        grid_spec=pltpu.PrefetchScalarGridSpec(
            num_scalar_prefetch=1,
            grid=(B, 2),
            in_specs=[
                pl.BlockSpec((1, S, F, D), lambda b, p, seg: (b, 0, 0, 0)),
                pl.BlockSpec((1, 1, CD), lambda b, p, seg: (b, 0, 0)),
                pl.BlockSpec((1, 1, CD), lambda b, p, seg: (b, 0, 0)),
                pl.BlockSpec((D, SH), const(0, 0)),
                pl.BlockSpec((1, SH), const(0, 0)),
                pl.BlockSpec(W_t1.shape, const(0, 0)),
                pl.BlockSpec((1, W_t1.shape[1]), const(0, 0)),
                pl.BlockSpec(W_t2.shape, const(0, 0)),
                pl.BlockSpec((1, TN), const(0, 0)),
                pl.BlockSpec((TN, TE), const(0, 0)),
                pl.BlockSpec(W_c.shape, const(0, 0)),
                pl.BlockSpec((D, 1), const(0, 0)),
            ],
            out_specs=pl.BlockSpec((1, S, F, 1), lambda b, p, seg: (b, 0, 0, 0)),
            scratch_shapes=[
                pltpu.VMEM((D, TN), jnp.float32),
                pltpu.VMEM((1, TN), jnp.float32),
            ],
        ),
        out_shape=jax.ShapeDtypeStruct((B, S, F, 1), jnp.float32),
        compiler_params=pltpu.CompilerParams(
            dimension_semantics=("parallel", "arbitrary")),
    )(seg_len, batch, concept1.reshape(B, 1, CD), concept2.reshape(B, 1, CD),
      W_enc, b_enc.reshape(1, SH), W_t1, b_t1.reshape(1, -1),
      W_t2, b_t2.reshape(1, TN), topic_emb, W_c, b_c.reshape(D, 1))

    overall = overall.reshape(B, S, F)
    return (overall, overall)
